# Initial kernel scaffold; baseline (speedup 1.0000x reference)
import jax
import jax.numpy as jnp
from jax.experimental import pallas as pl


def kernel(tertiary, sequence, subgraph_indices, knn_noise, W_in, Wq, Wk, Wv, Wo, M1, b1, M2, b2, M3, b3, Cw, Cb, Ew, Eb):
    raise NotImplementedError("write your pallas kernel here")


# speedup vs baseline: 4.9133x; 4.9133x over previous
import functools

import jax
import jax.numpy as jnp
import numpy as np
from jax.experimental import pallas as pl
from jax.experimental.pallas import tpu as pltpu

B, L, K = 8, 512, 15
KP = 16
N = B * L
SIZE = 128
HEADS = 8
DH = 16
HIDDEN = 512
DEPTH = 3
MSG = SIZE + 29
MAXD = 20.0
KERNELS = 16
NEG = -1e30


def _normalize(v, eps):
    nrm = jnp.sqrt(v[0:1] * v[0:1] + v[1:2] * v[1:2] + v[2:3] * v[2:3])
    return v / (nrm + eps)


def _cross(a, b):
    ax, ay, az = a[0:1], a[1:2], a[2:3]
    bx, by, bz = b[0:1], b[1:2], b[2:3]
    return jnp.concatenate(
        [ay * bz - az * by, az * bx - ax * bz, ax * by - ay * bx], axis=0)


def _body(posT_ref, noise_ref, Win_ref, Wq_ref, Wkx_ref, Wkr_ref, Wvx_ref,
          Wvr_ref, Wo_ref, M1_ref, b1_ref, M2_ref, b2_ref, M3_ref, b3_ref,
          CwT_ref, Cb_ref, Ew_ref, Eb_ref, out_ref, fr_ref):
    b = pl.program_id(0)
    f32 = jnp.float32

    pos = posT_ref[:, :]
    pos_next = jnp.concatenate([pos[:, 1:], pos[:, N - 1:]], axis=1)
    d = pos_next - pos
    dn = _normalize(d, 1e-6)
    v1 = jnp.concatenate([dn[:, 0:1], dn[:, :N - 1]], axis=1)
    v2 = jnp.concatenate([dn[:, :N - 1], dn[:, N - 2:N - 1]], axis=1)
    bv = _normalize(v1 - v2, 1e-6)
    nv = _normalize(_cross(v1, v2), 1e-6)
    cv = _cross(bv, nv)

    fr_ref[0:3, :] = pos
    fr_ref[3:6, :] = bv
    fr_ref[6:9, :] = nv
    fr_ref[9:12, :] = cv
    off = pl.multiple_of(b * L, L)
    ST = fr_ref[:, pl.ds(off, L)]
    pos_b = ST[0:3, :]
    ri0 = jax.lax.broadcasted_iota(jnp.int32, (L, L), 0)
    ci0 = jax.lax.broadcasted_iota(jnp.int32, (L, L), 1)
    eye = (ri0 == ci0).astype(f32)
    S = jax.lax.dot_general(eye, ST, (((1,), (1,)), ((), ())),
                            preferred_element_type=f32)

    d2 = jnp.zeros((L, L), f32)
    for c in range(3):
        D = S[:, c:c + 1] - pos_b[c:c + 1, :]
        d2 = d2 + D * D
    closeness = -jnp.sqrt(d2 + 1e-8) + 3.0 * noise_ref[0]

    lane = ci0
    cwork = closeness
    idx_cols = []
    for _ in range(KP):
        m = jnp.max(cwork, axis=1, keepdims=True)
        cand = jnp.where(cwork >= m, lane, L)
        pick = jnp.min(cand, axis=1, keepdims=True)
        idx_cols.append(pick)
        cwork = jnp.where(lane == pick, NEG, cwork)

    mu = jax.lax.broadcasted_iota(jnp.int32, (L, KERNELS), 1).astype(f32) * (
        MAXD / (KERNELS - 1))
    sigma = MAXD / KERNELS
    rows_f = jax.lax.broadcasted_iota(jnp.int32, (L, 1), 0).astype(f32)
    relf_slabs = []
    for k in range(KP):
        oh = (lane == idx_cols[k]).astype(f32)
        slab = jax.lax.dot_general(oh, S, (((1,), (0,)), ((), ())),
                                   preferred_element_type=f32)
        rel = slab[:, 0:3] - S[:, 0:3]
        dist = jnp.sqrt(rel[:, 0:1] ** 2 + rel[:, 1:2] ** 2 +
                        rel[:, 2:3] ** 2 + 1e-8)
        direction = rel / (dist + 1e-6)
        rbf = jnp.exp(-(((dist - mu) / sigma) ** 2))
        ro = []
        for bb in range(3):
            for cc in range(3):
                acc = jnp.zeros((L, 1), f32)
                for v in range(3):
                    acc = acc + S[:, 3 + 3 * v + bb:4 + 3 * v + bb] * \
                        slab[:, 3 + 3 * v + cc:4 + 3 * v + cc]
                ro.append(acc)
        ri = (idx_cols[k].astype(f32) - rows_f) * (1.0 / L)
        relf_slabs.append(jnp.concatenate(
            [rbf, direction] + ro + [ri, jnp.zeros((L, 3), f32)], axis=1))
    relf = jnp.concatenate(relf_slabs, axis=0)

    hsel = (jax.lax.broadcasted_iota(jnp.int32, (SIZE, HEADS), 0) // DH ==
            jax.lax.broadcasted_iota(jnp.int32, (SIZE, HEADS), 1)
            ).astype(f32) * (1.0 / np.sqrt(DH))
    esel = (jax.lax.broadcasted_iota(jnp.int32, (HEADS, SIZE), 0) ==
            jax.lax.broadcasted_iota(jnp.int32, (HEADS, SIZE), 1) // DH
            ).astype(f32)

    def mm(a, w):
        return jax.lax.dot_general(a, w, (((1,), (0,)), ((), ())),
                                   preferred_element_type=f32)

    x = jnp.broadcast_to(jnp.sum(Win_ref[:, :], axis=0, keepdims=True),
                         (L, SIZE))

    for l in range(DEPTH):
        q = mm(x, Wq_ref[l])
        qt = jnp.concatenate([q] * KP, axis=0)
        xg = jnp.concatenate(
            [mm((lane == idx_cols[k]).astype(f32), x) for k in range(KP)],
            axis=0)
        kk = mm(xg, Wkx_ref[l]) + mm(relf, Wkr_ref[l])
        vv = mm(xg, Wvx_ref[l]) + mm(relf, Wvr_ref[l])
        logits = mm(qt * kk, hsel)
        lg = [logits[k * L:(k + 1) * L, :] for k in range(K)]
        m = lg[0]
        for k in range(1, K):
            m = jnp.maximum(m, lg[k])
        es = [jnp.exp(lg[k] - m) for k in range(K)]
        ssum = es[0]
        for k in range(1, K):
            ssum = ssum + es[k]
        inv = 1.0 / ssum
        attn = jnp.zeros((L, SIZE), f32)
        for k in range(K):
            attn = attn + mm(es[k] * inv, esel) * vv[k * L:(k + 1) * L, :]
        x = x + mm(attn, Wo_ref[l])
        h = jnp.maximum(mm(x, M1_ref[l]) + b1_ref[l:l + 1, :], 0.0)
        h = jnp.maximum(mm(h, M2_ref[l]) + b2_ref[l:l + 1, :], 0.0)
        x = x + mm(h, M3_ref[l]) + b3_ref[l:l + 1, :]

    t = x
    tlen = L
    for i in range(4):
        zrow = jnp.zeros((1, SIZE), f32)
        sh0 = jnp.concatenate([zrow, t[:tlen - 1, :]], axis=0)
        sh2 = jnp.concatenate([t[1:, :], zrow], axis=0)
        c = mm(sh0, CwT_ref[i, 0]) + mm(t, CwT_ref[i, 1]) + mm(sh2, CwT_ref[i, 2])
        c = c + Cb_ref[i:i + 1, :]
        c = jnp.where(c > 0, c, 0.01 * c)
        t = t + c
        tlen //= 2
        r = t.reshape(tlen, 2, SIZE)
        t = jnp.maximum(r[:, 0, :], r[:, 1, :])

    pooled = jnp.sum(t, axis=0, keepdims=True)
    out_ref[0, :, :] = mm(pooled, Ew_ref[:, :]) + Eb_ref[:, :]


@jax.jit
def kernel(tertiary, sequence, subgraph_indices, knn_noise, W_in, Wq, Wk, Wv,
           Wo, M1, b1, M2, b2, M3, b3, Cw, Cb, Ew, Eb):
    del sequence, subgraph_indices
    f32 = jnp.float32
    posT = tertiary[:, 1, :].T.astype(f32)

    def split_msg(W):
        Wx = W[:, :SIZE, :]
        Wr = jnp.concatenate([
            W[:, 131:147, :], W[:, 128:131, :], W[:, 147:156, :],
            W[:, 156:157, :], jnp.zeros((DEPTH, 3, SIZE), f32)], axis=1)
        return Wx, Wr

    Wkx, Wkr = split_msg(Wk)
    Wvx, Wvr = split_msg(Wv)
    CwT = Cw.transpose(0, 3, 2, 1)
    Eb2 = Eb.reshape(1, 1)

    full = lambda shape: pl.BlockSpec(shape, lambda b: (0,) * len(shape))
    in_specs = [
            full((3, N)),
            pl.BlockSpec((1, L, L), lambda b: (b, 0, 0)),
            full((27, SIZE)),
            full((DEPTH, SIZE, HEADS * DH)),
            full((DEPTH, SIZE, SIZE)),
            full((DEPTH, 32, SIZE)),
            full((DEPTH, SIZE, SIZE)),
            full((DEPTH, 32, SIZE)),
            full((DEPTH, HEADS * DH, SIZE)),
            full((DEPTH, SIZE, HIDDEN)),
            full((DEPTH, HIDDEN)),
            full((DEPTH, HIDDEN, HIDDEN)),
            full((DEPTH, HIDDEN)),
            full((DEPTH, HIDDEN, SIZE)),
            full((DEPTH, SIZE)),
            full((4, 3, SIZE, SIZE)),
            full((4, SIZE)),
            full((SIZE, 1)),
            full((1, 1)),
        ]
    out = pl.pallas_call(
        _body,
        grid=(B,),
        in_specs=in_specs,
        out_specs=pl.BlockSpec((1, 1, 1), lambda b: (b, 0, 0)),
        out_shape=jax.ShapeDtypeStruct((B, 1, 1), f32),
        scratch_shapes=[pltpu.VMEM((12, N), f32)],
    )(posT, knn_noise, W_in, Wq, Wkx, Wkr, Wvx, Wvr, Wo, M1, b1, M2, b2, M3,
      b3, CwT, Cb, Ew, Eb2)
    return out.reshape(B, 1)
